# baseline (speedup 1.0000x reference)
import jax
import jax.numpy as jnp
from jax import lax
from jax.experimental import pallas as pl
from jax.experimental.pallas import tpu as pltpu

from jax.experimental.pallas import tpu_sc as plsc

T, D, E, K, F = 2048, 2048, 8, 2, 768
P = T * K
BT = 256
RT = P // BT
VMAX = RT + E - 1
BF = 256


def _router_body(x_ref, gate_w_ref, pos_ref, w_ref, off_ref):
    logits = jnp.dot(x_ref[...], gate_w_ref[...].T, preferred_element_type=jnp.float32)
    m = jnp.max(logits, axis=1, keepdims=True)
    p = jnp.exp(logits - m)
    p = p / jnp.sum(p, axis=1, keepdims=True)
    iota = lax.broadcasted_iota(jnp.int32, (T, E), 1)
    m1 = jnp.max(p, axis=1, keepdims=True)
    i1 = jnp.min(jnp.where(p == m1, iota, E), axis=1, keepdims=True)
    mask1 = iota == i1
    p2 = jnp.where(mask1, -1.0, p)
    m2 = jnp.max(p2, axis=1, keepdims=True)
    i2 = jnp.min(jnp.where(p2 == m2, iota, E), axis=1, keepdims=True)
    mask2 = iota == i2
    s = m1 + m2
    h = (mask1 | mask2).astype(jnp.float32)
    ri = lax.broadcasted_iota(jnp.int32, (T, T), 0)
    ci = lax.broadcasted_iota(jnp.int32, (T, T), 1)
    ltri = (ci < ri).astype(jnp.float32)
    excl = jnp.dot(ltri, h, preferred_element_type=jnp.float32)
    counts = jnp.sum(h, axis=0, keepdims=True)
    ue = (lax.broadcasted_iota(jnp.int32, (E, E), 0)
          < lax.broadcasted_iota(jnp.int32, (E, E), 1)).astype(jnp.float32)
    off = jnp.dot(counts, ue, preferred_element_type=jnp.float32,
                  precision=lax.Precision.HIGHEST)
    posg = off + excl
    pos1 = jnp.sum(jnp.where(mask1, posg, 0.0), axis=1, keepdims=True)
    pos2 = jnp.sum(jnp.where(mask2, posg, 0.0), axis=1, keepdims=True)
    pos_ref[...] = jnp.round(jnp.concatenate([pos1, pos2], axis=1)).astype(jnp.int32)
    w_ref[...] = jnp.concatenate([m1 / s, m2 / s], axis=1)
    off_ref[...] = jnp.round(off).astype(jnp.int32)


def _router(x, gate_w):
    return pl.pallas_call(
        _router_body,
        in_specs=[
            pl.BlockSpec((T, D), lambda: (0, 0)),
            pl.BlockSpec((E, D), lambda: (0, 0)),
        ],
        out_specs=[
            pl.BlockSpec((T, K), lambda: (0, 0)),
            pl.BlockSpec((T, K), lambda: (0, 0)),
            pl.BlockSpec((1, E), lambda: (0, 0)),
        ],
        out_shape=[
            jax.ShapeDtypeStruct((T, K), jnp.int32),
            jax.ShapeDtypeStruct((T, K), jnp.float32),
            jax.ShapeDtypeStruct((1, E), jnp.int32),
        ],
    )(x, gate_w)


def _visit_metadata(off):
    off_f = off.reshape(E)
    offp = jnp.concatenate([off_f, jnp.array([P], jnp.int32)])
    counts = offp[1:] - offp[:-1]
    start_tile = off_f // BT
    end_tile = (offp[1:] + BT - 1) // BT
    ntiles = jnp.where(counts > 0, end_tile - start_tile, 0)
    c_incl = jnp.cumsum(ntiles)
    vbase = c_incl - ntiles
    total_v = c_incl[-1]
    v_iota = jnp.arange(VMAX, dtype=jnp.int32)
    e_of_v = jnp.sum((c_incl[None, :] <= v_iota[:, None]).astype(jnp.int32), axis=1)
    valid = v_iota < total_v
    e_cl = jnp.minimum(e_of_v, E - 1)
    tile_v = start_tile[e_cl] + (v_iota - vbase[e_cl])
    tile_v = jnp.clip(tile_v, 0, RT - 1)
    tile_v = jnp.where(valid, tile_v, RT - 1)
    first = jnp.concatenate(
        [jnp.ones((1,), jnp.int32),
         (tile_v[1:] != tile_v[:-1]).astype(jnp.int32)])
    lo_g = jnp.maximum(offp[e_cl], tile_v * BT)
    hi_g = jnp.minimum(offp[e_cl + 1], tile_v * BT + BT)
    mlo = jnp.where(valid, lo_g - tile_v * BT, 0)
    mhi = jnp.where(valid, hi_g - tile_v * BT, 0)
    return tile_v.astype(jnp.int32), e_cl.astype(jnp.int32), first, mlo, mhi


def _gg_body(tiles_r, exps_r, firsts_r, mlo_r, mhi_r,
             xs_ref, wg_ref, wu_ref, wd_ref, ys_ref):
    fb = pl.program_id(0)
    v = pl.program_id(1)
    x = xs_ref[...]
    g = jnp.dot(x, wg_ref[0].astype(jnp.bfloat16).T, preferred_element_type=jnp.float32)
    u = jnp.dot(x, wu_ref[0].astype(jnp.bfloat16).T, preferred_element_type=jnp.float32)
    rows = lax.broadcasted_iota(jnp.int32, (BT, 1), 0)
    mask = (rows >= mlo_r[v]) & (rows < mhi_r[v])
    a = (jax.nn.silu(g) * u * jnp.where(mask, 1.0, 0.0)).astype(jnp.bfloat16)
    part = jnp.dot(a, wd_ref[0].astype(jnp.bfloat16).T, preferred_element_type=jnp.float32)
    first_step = (fb == 0) & (firsts_r[v] == 1)
    row0 = tiles_r[v] * BT

    @pl.when(first_step)
    def _init():
        ys_ref[pl.ds(row0, BT), :] = part

    @pl.when(~first_step)
    def _acc():
        ys_ref[pl.ds(row0, BT), :] += part


def _group_gemm(tiles, exps, firsts, mlo, mhi, xs16, w_gate, w_up, w_down):
    grid_spec = pltpu.PrefetchScalarGridSpec(
        num_scalar_prefetch=5,
        grid=(F // BF, VMAX),
        in_specs=[
            pl.BlockSpec((BT, D), lambda fb, v, t, e, fi, lo, hi: (t[v], 0)),
            pl.BlockSpec((1, BF, D), lambda fb, v, t, e, fi, lo, hi: (e[v], fb, 0)),
            pl.BlockSpec((1, BF, D), lambda fb, v, t, e, fi, lo, hi: (e[v], fb, 0)),
            pl.BlockSpec((1, D, BF), lambda fb, v, t, e, fi, lo, hi: (e[v], 0, fb)),
        ],
        out_specs=pl.BlockSpec((P, D), lambda fb, v, t, e, fi, lo, hi: (0, 0)),
    )
    return pl.pallas_call(
        _gg_body,
        grid_spec=grid_spec,
        out_shape=jax.ShapeDtypeStruct((P, D), jnp.float32),
        compiler_params=pltpu.CompilerParams(
            dimension_semantics=("arbitrary", "arbitrary"),
        ),
    )(tiles, exps, firsts, mlo, mhi, xs16, w_gate, w_up, w_down)


DW = D // 2
NW = 32
TPW = T // NW
CH = 4
CT = TPW // CH


def _c1_body(x_ref, posA_ref, posB_ref, xs_ref, xbuf, idxA, idxB, semg):
    wid = lax.axis_index("s") * 2 + lax.axis_index("c")
    for c in range(CH):
        tok0 = wid * TPW + c * CT
        pltpu.sync_copy(x_ref.at[pl.ds(tok0, CT)], xbuf)
        pltpu.sync_copy(posA_ref.at[wid, c], idxA)
        pltpu.sync_copy(posB_ref.at[wid, c], idxB)
        cp1 = pltpu.async_copy(xbuf, xs_ref.at[idxA], semg)
        cp2 = pltpu.async_copy(xbuf, xs_ref.at[idxB], semg)
        cp1.wait()
        cp2.wait()


def _scatter_sc(x_i, posA, posB):
    mesh = plsc.VectorSubcoreMesh(core_axis_name="c", subcore_axis_name="s")
    f = pl.kernel(
        _c1_body,
        out_type=jax.ShapeDtypeStruct((P, DW), jnp.int32),
        mesh=mesh,
        scratch_types=[
            pltpu.VMEM((CT, DW), jnp.int32),
            pltpu.VMEM((16,), jnp.int32),
            pltpu.VMEM((16,), jnp.int32),
            pltpu.SemaphoreType.DMA,
        ],
    )
    return f(x_i, posA, posB)


def _c3_body(ys_ref, posA_ref, posB_ref, wA_ref, wB_ref, out_ref,
             bufA, bufB, idxA, idxB, wbufA, wbufB, sem):
    wid = lax.axis_index("s") * 2 + lax.axis_index("c")
    for c in range(CH):
        tok0 = wid * TPW + c * CT
        pltpu.sync_copy(posA_ref.at[wid, c], idxA)
        pltpu.sync_copy(posB_ref.at[wid, c], idxB)
        pltpu.sync_copy(wA_ref.at[wid, c], wbufA)
        pltpu.sync_copy(wB_ref.at[wid, c], wbufB)
        cp1 = pltpu.async_copy(ys_ref.at[idxA], bufA, sem)
        cp2 = pltpu.async_copy(ys_ref.at[idxB], bufB, sem)
        cp1.wait()
        cp2.wait()
        wvA = wbufA[...]
        wvB = wbufB[...]
        for r in range(CT):
            wa = wvA[r]
            wb = wvB[r]
            def _add(i, _, r=r, wa=wa, wb=wb):
                sl = pl.ds(i * 16, 16)
                bufA[r, sl] = bufA[r, sl] * wa + bufB[r, sl] * wb
                return _
            lax.fori_loop(0, D // 16, _add, 0)
        pltpu.sync_copy(bufA, out_ref.at[pl.ds(tok0, CT)])


def _combine_sc(ys, posA, posB, wA, wB):
    mesh = plsc.VectorSubcoreMesh(core_axis_name="c", subcore_axis_name="s")
    f = pl.kernel(
        _c3_body,
        out_type=jax.ShapeDtypeStruct((T, D), jnp.float32),
        mesh=mesh,
        scratch_types=[
            pltpu.VMEM((CT, D), jnp.float32),
            pltpu.VMEM((CT, D), jnp.float32),
            pltpu.VMEM((16,), jnp.int32),
            pltpu.VMEM((16,), jnp.int32),
            pltpu.VMEM((16,), jnp.float32),
            pltpu.VMEM((16,), jnp.float32),
            pltpu.SemaphoreType.DMA,
        ],
    )
    return f(ys, posA, posB, wA, wB)


def kernel(hidden_states, gate_w, w_gate, w_up, w_down):
    x = hidden_states.reshape(-1, D)
    pos, wsel, off = _router(x, gate_w)
    tiles, exps, firsts, mlo, mhi = _visit_metadata(off)
    x16 = x.astype(jnp.bfloat16)
    posA = pos[:, 0].reshape(NW, CH, CT)
    posB = pos[:, 1].reshape(NW, CH, CT)
    wA = wsel[:, 0].reshape(NW, CH, CT)
    wB = wsel[:, 1].reshape(NW, CH, CT)
    x_i = lax.bitcast_convert_type(x16.reshape(T, DW, 2), jnp.int32)
    xs_i = _scatter_sc(x_i, posA, posB)
    xs16 = lax.bitcast_convert_type(xs_i, jnp.bfloat16).reshape(P, D)
    ys = _group_gemm(tiles, exps, firsts, mlo, mhi, xs16,
                     w_gate, w_up, w_down)
    out = _combine_sc(ys, posA, posB, wA, wB)
    return out.reshape(hidden_states.shape)
